# Initial kernel scaffold; baseline (speedup 1.0000x reference)
import functools

import jax
import jax.numpy as jnp
from jax import lax
from jax.experimental import pallas as pl
from jax.experimental.pallas import tpu as pltpu
from jax.experimental.pallas import tpu_sc as plsc

_N = 10000
_E = 320000
_D = 128
_G = 64
_EPS = 1e-5

_NC = 2
_NS = 16
_NW = _NC * _NS
_EW = _E // _NW
_CH = 128
_NFULL = _EW // _CH
_TAIL = _EW - _NFULL * _CH
_RPT = _N // _NS


def _sc_scatter_body(h_hbm, src_hbm, dst_hbm, zeros_hbm, out_hbm,
                     idx_s, idx_d, rows, tidx_s, tidx_d, trows, acc, sem):
    c = lax.axis_index("c")
    s = lax.axis_index("s")
    wid = c * _NS + s
    base = wid * _EW
    r0 = s * _RPT

    pltpu.sync_copy(zeros_hbm.at[pl.ds(r0, _RPT), :], acc.at[pl.ds(r0, _RPT), :])
    plsc.subcore_barrier()

    def step(i, carry):
        off = base + i * _CH
        pltpu.sync_copy(src_hbm.at[pl.ds(off, _CH)], idx_s)
        pltpu.sync_copy(dst_hbm.at[pl.ds(off, _CH)], idx_d)
        pltpu.async_copy(h_hbm.at[idx_s], rows, sem).wait()
        pltpu.sync_copy(rows, acc.at[idx_d], add=True)
        return carry

    lax.fori_loop(0, _NFULL, step, 0)

    toff = base + _NFULL * _CH
    pltpu.sync_copy(src_hbm.at[pl.ds(toff, _TAIL)], tidx_s)
    pltpu.sync_copy(dst_hbm.at[pl.ds(toff, _TAIL)], tidx_d)
    pltpu.async_copy(h_hbm.at[tidx_s], trows, sem).wait()
    pltpu.sync_copy(trows, acc.at[tidx_d], add=True)

    plsc.subcore_barrier()
    pltpu.sync_copy(acc.at[pl.ds(r0, _RPT), :], out_hbm.at[c, pl.ds(r0, _RPT), :])


_sc_scatter = pl.kernel(
    _sc_scatter_body,
    out_type=jax.ShapeDtypeStruct((_NC, _N, _D), jnp.float32),
    mesh=plsc.VectorSubcoreMesh(core_axis_name="c", subcore_axis_name="s"),
    scratch_types=[
        pltpu.VMEM((_CH,), jnp.int32),
        pltpu.VMEM((_CH,), jnp.int32),
        pltpu.VMEM((_CH, _D), jnp.float32),
        pltpu.VMEM((_TAIL,), jnp.int32),
        pltpu.VMEM((_TAIL,), jnp.int32),
        pltpu.VMEM((_TAIL, _D), jnp.float32),
        pltpu.VMEM_SHARED((_N, _D), jnp.float32),
        pltpu.SemaphoreType.DMA,
    ],
)


def _tc_layer_body(final, h_ref, aggr_ref, batch_ref,
                   vw_ref, vb_ref, aw_ref, ab_ref, rw_ref, rb_ref,
                   g_ref, b_ref, lw_ref, lb_ref, out_ref):
    h = h_ref[...]
    aggr = aggr_ref[0] + aggr_ref[1]
    onehot = (batch_ref[...] ==
              lax.broadcasted_iota(jnp.int32, (_N, _G), 1)).astype(jnp.float32)
    pooled = lax.dot_general(onehot, h, (((0,), (0,)), ((), ())),
                             preferred_element_type=jnp.float32)
    pr = jnp.dot(pooled, rw_ref[...], preferred_element_type=jnp.float32)
    comb = (jnp.dot(h, vw_ref[...], preferred_element_type=jnp.float32)
            + jnp.dot(aggr, aw_ref[...], preferred_element_type=jnp.float32)
            + jnp.dot(onehot, pr, preferred_element_type=jnp.float32)
            + vb_ref[...] + ab_ref[...] + rb_ref[...])
    hr = jnp.maximum(comb, 0.0)
    mean = jnp.mean(hr, axis=0, keepdims=True)
    var = jnp.mean((hr - mean) * (hr - mean), axis=0, keepdims=True)
    hn = (hr - mean) * lax.rsqrt(var + _EPS) * g_ref[...] + b_ref[...]
    if final:
        out_ref[...] = (jnp.dot(hn, lw_ref[...],
                                preferred_element_type=jnp.float32)
                        + lb_ref[...])
    else:
        out_ref[...] = hn


def _tc_layer(final, h, aggr, batch_col, vw, vb, aw, ab, rw, rb, g, b, lw, lb):
    return pl.pallas_call(
        functools.partial(_tc_layer_body, final),
        out_shape=jax.ShapeDtypeStruct((_N, lw.shape[1] if final else _D),
                                       jnp.float32),
    )(h, aggr, batch_col, vw, vb.reshape(1, -1), aw, ab.reshape(1, -1),
      rw, rb.reshape(1, -1), g.reshape(1, -1), b.reshape(1, -1),
      lw, lb.reshape(1, -1))


def kernel(x, edge_index, batch,
           V0w, V0b, A0w, A0b, R0w, R0b, bn0_g, bn0_b,
           V1w, V1b, A1w, A1b, R1w, R1b, bn1_g, bn1_b,
           lin_w, lin_b):
    src = edge_index[0]
    dst = edge_index[1]
    zeros = jnp.zeros((_N, _D), jnp.float32)
    batch_col = batch.reshape(_N, 1)

    aggr0 = _sc_scatter(x, src, dst, zeros)
    h1 = _tc_layer(False, x, aggr0, batch_col,
                   V0w, V0b, A0w, A0b, R0w, R0b, bn0_g, bn0_b, lin_w, lin_b)
    aggr1 = _sc_scatter(h1, src, dst, zeros)
    out = _tc_layer(True, h1, aggr1, batch_col,
                    V1w, V1b, A1w, A1b, R1w, R1b, bn1_g, bn1_b, lin_w, lin_b)
    return out


# speedup vs baseline: 5.7183x; 5.7183x over previous
import functools

import jax
import jax.numpy as jnp
from jax import lax
from jax.experimental import pallas as pl
from jax.experimental.pallas import tpu as pltpu
from jax.experimental.pallas import tpu_sc as plsc

_N = 10000
_E = 320000
_D = 128
_G = 64
_EPS = 1e-5

_NC = 2
_NS = 16
_NW = _NC * _NS
_EW = _E // _NW
_CH = 128
_NFULL = _EW // _CH
_TAIL = _EW - _NFULL * _CH
_NP = 10240
_RPT = _NP // _NS


def _sc_scatter_body(h_hbm, src_hbm, dst_hbm, zeros_hbm, out_hbm,
                     idx_s, idx_d, rows, tidx_s, tidx_d, trows, acc, sem):
    c = lax.axis_index("c")
    s = lax.axis_index("s")
    wid = c * _NS + s
    base = wid * _EW
    r0 = s * _RPT

    pltpu.sync_copy(zeros_hbm.at[pl.ds(r0, _RPT), :], acc.at[pl.ds(r0, _RPT), :])
    plsc.subcore_barrier()

    def step(i, carry):
        off = base + i * _CH
        pltpu.sync_copy(src_hbm.at[pl.ds(off, _CH)], idx_s)
        pltpu.sync_copy(dst_hbm.at[pl.ds(off, _CH)], idx_d)
        pltpu.async_copy(h_hbm.at[idx_s], rows, sem).wait()
        pltpu.sync_copy(rows, acc.at[idx_d], add=True)
        return carry

    lax.fori_loop(0, _NFULL, step, 0)

    toff = base + _NFULL * _CH
    pltpu.sync_copy(src_hbm.at[pl.ds(toff, _TAIL)], tidx_s)
    pltpu.sync_copy(dst_hbm.at[pl.ds(toff, _TAIL)], tidx_d)
    pltpu.async_copy(h_hbm.at[tidx_s], trows, sem).wait()
    pltpu.sync_copy(trows, acc.at[tidx_d], add=True)

    plsc.subcore_barrier()
    pltpu.sync_copy(acc.at[pl.ds(r0, _RPT), :], out_hbm.at[c, pl.ds(r0, _RPT), :])


@functools.cache
def _get_sc_scatter():
    return pl.kernel(
        _sc_scatter_body,
        out_type=jax.ShapeDtypeStruct((_NC, _NP, _D), jnp.float32),
        mesh=plsc.VectorSubcoreMesh(core_axis_name="c", subcore_axis_name="s"),
        scratch_types=[
            pltpu.VMEM((_CH,), jnp.int32),
            pltpu.VMEM((_CH,), jnp.int32),
            pltpu.VMEM((_CH, _D), jnp.float32),
            pltpu.VMEM((_TAIL,), jnp.int32),
            pltpu.VMEM((_TAIL,), jnp.int32),
            pltpu.VMEM((_TAIL, _D), jnp.float32),
            pltpu.VMEM_SHARED((_NP, _D), jnp.float32),
            pltpu.SemaphoreType.DMA,
        ],
    )


def _tc_layer_body(final, h_ref, aggr_ref, batch_ref,
                   vw_ref, vb_ref, aw_ref, ab_ref, rw_ref, rb_ref,
                   g_ref, b_ref, lw_ref, lb_ref, out_ref):
    h = h_ref[...]
    aggr = (aggr_ref[0] + aggr_ref[1])[:_N]
    onehot = (batch_ref[...] ==
              lax.broadcasted_iota(jnp.int32, (_N, _G), 1)).astype(jnp.float32)
    pooled = lax.dot_general(onehot, h, (((0,), (0,)), ((), ())),
                             preferred_element_type=jnp.float32)
    pr = jnp.dot(pooled, rw_ref[...], preferred_element_type=jnp.float32)
    comb = (jnp.dot(h, vw_ref[...], preferred_element_type=jnp.float32)
            + jnp.dot(aggr, aw_ref[...], preferred_element_type=jnp.float32)
            + jnp.dot(onehot, pr, preferred_element_type=jnp.float32)
            + vb_ref[...] + ab_ref[...] + rb_ref[...])
    hr = jnp.maximum(comb, 0.0)
    mean = jnp.mean(hr, axis=0, keepdims=True)
    var = jnp.mean((hr - mean) * (hr - mean), axis=0, keepdims=True)
    hn = (hr - mean) * lax.rsqrt(var + _EPS) * g_ref[...] + b_ref[...]
    if final:
        out_ref[...] = (jnp.dot(hn, lw_ref[...],
                                preferred_element_type=jnp.float32)
                        + lb_ref[...])
    else:
        out_ref[...] = hn


def _tc_layer(final, h, aggr, batch_col, vw, vb, aw, ab, rw, rb, g, b, lw, lb):
    return pl.pallas_call(
        functools.partial(_tc_layer_body, final),
        out_shape=jax.ShapeDtypeStruct((_N, lw.shape[1] if final else _D),
                                       jnp.float32),
    )(h, aggr, batch_col, vw, vb.reshape(1, -1), aw, ab.reshape(1, -1),
      rw, rb.reshape(1, -1), g.reshape(1, -1), b.reshape(1, -1),
      lw, lb.reshape(1, -1))


def kernel(x, edge_index, batch,
           V0w, V0b, A0w, A0b, R0w, R0b, bn0_g, bn0_b,
           V1w, V1b, A1w, A1b, R1w, R1b, bn1_g, bn1_b,
           lin_w, lin_b):
    src = edge_index[0]
    dst = edge_index[1]
    zeros = jnp.zeros((_NP, _D), jnp.float32)
    batch_col = batch.reshape(_N, 1)

    sc_scatter = _get_sc_scatter()
    aggr0 = sc_scatter(x, src, dst, zeros)
    h1 = _tc_layer(False, x, aggr0, batch_col,
                   V0w, V0b, A0w, A0b, R0w, R0b, bn0_g, bn0_b, lin_w, lin_b)
    aggr1 = sc_scatter(h1, src, dst, zeros)
    out = _tc_layer(True, h1, aggr1, batch_col,
                    V1w, V1b, A1w, A1b, R1w, R1b, bn1_g, bn1_b, lin_w, lin_b)
    return out
